# baseline (speedup 1.0000x reference)
import functools

import jax
import jax.numpy as jnp
from jax.experimental import pallas as pl
from jax.experimental.pallas import tpu as pltpu

B = 1024
K = 100000
D = 16
KT = 2048
KPAD = 100352
NSTEPS = KPAD // KT

_KS0 = 0
_KS1 = 42
_KS2 = 0x1BD11BDA ^ 42
_ROTS = ((13, 15, 26, 6), (17, 29, 16, 24))
_TINY = 1.1754943508222875e-38


def _threefry_gumbel(flat_idx_u32):
    x0 = jnp.zeros_like(flat_idx_u32)
    x1 = flat_idx_u32 + jnp.uint32(_KS1)
    ks = (_KS0, _KS1, _KS2)
    for i in range(5):
        for r in _ROTS[i % 2]:
            x0 = x0 + x1
            x1 = (x1 << r) | (x1 >> (32 - r))
            x1 = x1 ^ x0
        x0 = x0 + jnp.uint32(ks[(i + 1) % 3])
        x1 = x1 + jnp.uint32((ks[(i + 2) % 3] + i + 1) & 0xFFFFFFFF)
    bits = x0 ^ x1
    fl = jax.lax.bitcast_convert_type((bits >> 9) | jnp.uint32(0x3F800000),
                                      jnp.float32) - jnp.float32(1.0)
    tiny = jnp.float32(_TINY)
    u = jnp.maximum(tiny, fl * (jnp.float32(1.0) - tiny) + tiny)
    return -jnp.log(-jnp.log(u))


def _mixture_sample_kernel(xs_ref, m_ref, a_ref, b_ref, c_ref, v_ref,
                           out_ref, bv_ref, bi_ref):
    j = pl.program_id(0)

    @pl.when(j == 0)
    def _init():
        bv_ref[...] = jnp.full((B, 1), -jnp.inf, jnp.float32)
        bi_ref[...] = jnp.zeros((B, 1), jnp.int32)

    mm = jax.lax.dot_general(xs_ref[...], m_ref[...],
                             dimension_numbers=(((1,), (1,)), ((), ())),
                             preferred_element_type=jnp.float32)
    sq = (a_ref[...] + b_ref[0]) - 2.0 * mm
    logp = (jnp.float32(-0.5) * sq) / v_ref[0, 0] - c_ref[0, 0]

    row = jax.lax.broadcasted_iota(jnp.int32, (B, KT), 0).astype(jnp.uint32)
    col = jax.lax.broadcasted_iota(jnp.int32, (B, KT), 1)
    kg = j * KT + col
    flat = row * jnp.uint32(K) + kg.astype(jnp.uint32)
    cand = logp + _threefry_gumbel(flat)
    cand = jnp.where(kg < K, cand, -jnp.inf)

    tmax = jnp.max(cand, axis=1, keepdims=True)
    targ = jnp.argmax(cand, axis=1).astype(jnp.int32).reshape(B, 1) + j * KT
    upd = tmax > bv_ref[...]
    bi_ref[...] = jnp.where(upd, targ, bi_ref[...])
    bv_ref[...] = jnp.where(upd, tmax, bv_ref[...])

    @pl.when(j == NSTEPS - 1)
    def _done():
        out_ref[...] = bi_ref[...]


@jax.jit
def kernel(xs, means, cov):
    var = cov[0]
    a = jnp.sum(xs * xs, axis=1, keepdims=True)
    bv = jnp.sum(means * means, axis=1)
    cterm = (0.5 * D) * jnp.log(2.0 * jnp.pi * var)
    mpad = jnp.pad(means, ((0, KPAD - K), (0, 0)))
    bvpad = jnp.pad(bv, (0, KPAD - K)).reshape(NSTEPS, 1, KT)

    out = pl.pallas_call(
        _mixture_sample_kernel,
        grid=(NSTEPS,),
        in_specs=[
            pl.BlockSpec((B, D), lambda j: (0, 0)),
            pl.BlockSpec((KT, D), lambda j: (j, 0)),
            pl.BlockSpec((B, 1), lambda j: (0, 0)),
            pl.BlockSpec((1, 1, KT), lambda j: (j, 0, 0)),
            pl.BlockSpec((1, 1), lambda j: (0, 0)),
            pl.BlockSpec((1, 1), lambda j: (0, 0)),
        ],
        out_specs=pl.BlockSpec((B, 1), lambda j: (0, 0)),
        out_shape=jax.ShapeDtypeStruct((B, 1), jnp.int32),
        scratch_shapes=[pltpu.VMEM((B, 1), jnp.float32),
                        pltpu.VMEM((B, 1), jnp.int32)],
        compiler_params=pltpu.CompilerParams(
            dimension_semantics=("arbitrary",)),
    )(xs, mpad, a, bvpad, cterm.reshape(1, 1), var.reshape(1, 1))
    return out.reshape(B)


# speedup vs baseline: 1.1261x; 1.1261x over previous
import jax
import jax.numpy as jnp
from jax.experimental import pallas as pl
from jax.experimental.pallas import tpu as pltpu

B = 1024
K = 100000
D = 16
KT = 2048
KPAD = 100352
NSTEPS = KPAD // KT

_KS1 = 42
_KS2 = 0x1BD11BDA ^ 42
_ROTS = ((13, 15, 26, 6), (17, 29, 16, 24))
_TINY = 1.1754943508222875e-38


def _threefry_gumbel(flat_idx_u32):
    x0 = jnp.zeros_like(flat_idx_u32)
    x1 = flat_idx_u32 + jnp.uint32(_KS1)
    ks = (0, _KS1, _KS2)
    for i in range(5):
        for r in _ROTS[i % 2]:
            x0 = x0 + x1
            x1 = (x1 << r) | (x1 >> (32 - r))
            x1 = x1 ^ x0
        x0 = x0 + jnp.uint32(ks[(i + 1) % 3])
        x1 = x1 + jnp.uint32((ks[(i + 2) % 3] + i + 1) & 0xFFFFFFFF)
    bits = x0 ^ x1
    fl = jax.lax.bitcast_convert_type((bits >> 9) | jnp.uint32(0x3F800000),
                                      jnp.float32) - jnp.float32(1.0)
    tiny = jnp.float32(_TINY)
    u = jnp.maximum(tiny, fl * (jnp.float32(1.0) - tiny) + tiny)
    return -jnp.log(-jnp.log(u))


def _mixture_sample_kernel(xs_ref, mt_ref, a_ref, bv_ref, c_ref, v_ref,
                           out_ref):
    xs = xs_ref[...]
    a = a_ref[...]
    c = c_ref[0, 0]
    var = v_ref[0, 0]
    row = jax.lax.broadcasted_iota(jnp.int32, (B, KT), 0)
    col = jax.lax.broadcasted_iota(jnp.int32, (B, KT), 1)
    base = (row * K + col).astype(jnp.uint32)

    def body(t, carry):
        bval, bidx = carry
        off = t * KT
        mt = mt_ref[:, pl.ds(pl.multiple_of(off, KT), KT)]
        bvt = bv_ref[:, pl.ds(pl.multiple_of(off, KT), KT)]
        mm = jax.lax.dot_general(xs, mt,
                                 dimension_numbers=(((1,), (0,)), ((), ())),
                                 preferred_element_type=jnp.float32)
        sq = (a + bvt) - 2.0 * mm
        logp = (jnp.float32(-0.5) * sq) / var - c
        cand = logp + _threefry_gumbel(base + off.astype(jnp.uint32))
        tmax = jnp.max(cand, axis=1, keepdims=True)
        targ = jnp.argmax(cand, axis=1).astype(jnp.int32).reshape(B, 1) + off
        upd = tmax > bval
        return (jnp.where(upd, tmax, bval), jnp.where(upd, targ, bidx))

    init = (jnp.full((B, 1), -jnp.inf, jnp.float32),
            jnp.zeros((B, 1), jnp.int32))
    _, bidx = jax.lax.fori_loop(0, NSTEPS, body, init)
    out_ref[...] = bidx


@jax.jit
def kernel(xs, means, cov):
    var = cov[0]
    a = jnp.sum(xs * xs, axis=1, keepdims=True)
    bv = jnp.sum(means * means, axis=1)
    cterm = (0.5 * D) * jnp.log(2.0 * jnp.pi * var)
    mt = jnp.pad(means, ((0, KPAD - K), (0, 0))).T
    bvpad = jnp.pad(bv, (0, KPAD - K),
                    constant_values=jnp.inf).reshape(1, KPAD)

    out = pl.pallas_call(
        _mixture_sample_kernel,
        in_specs=[
            pl.BlockSpec((B, D), lambda: (0, 0)),
            pl.BlockSpec((D, KPAD), lambda: (0, 0)),
            pl.BlockSpec((B, 1), lambda: (0, 0)),
            pl.BlockSpec((1, KPAD), lambda: (0, 0)),
            pl.BlockSpec((1, 1), lambda: (0, 0)),
            pl.BlockSpec((1, 1), lambda: (0, 0)),
        ],
        out_specs=pl.BlockSpec((B, 1), lambda: (0, 0)),
        out_shape=jax.ShapeDtypeStruct((B, 1), jnp.int32),
    )(xs, mt, a, bvpad, cterm.reshape(1, 1), var.reshape(1, 1))
    return out.reshape(B)
